# baseline (speedup 1.0000x reference)
import jax
import jax.numpy as jnp
from jax.experimental import pallas as pl

N_BLOCKS = 7


def _mrconv(x, edge_index, W, b):
    src = edge_index[0]
    dst = edge_index[1]
    msg = x[src] - x[dst]
    agg = jax.ops.segment_max(msg, dst, num_segments=x.shape[0])
    agg = jnp.where(jnp.isneginf(agg), 0.0, agg)
    h = jnp.concatenate([x, agg], axis=1)
    return jax.nn.relu(h @ W + b)


def kernel(x, edge_index, batch, head_W, head_b, blocks_W, blocks_b,
           fusion_W, fusion_b, pred1_W, pred1_b, pred2_W, pred2_b,
           pred3_W, pred3_b):
    feats = [_mrconv(x, edge_index, head_W, head_b)]
    for i in range(N_BLOCKS - 1):
        h = _mrconv(feats[-1], edge_index, blocks_W[i], blocks_b[i]) + feats[-1]
        feats.append(h)
    feats = jnp.concatenate(feats, axis=1)
    fusion = jnp.max(jax.nn.relu(feats @ fusion_W + fusion_b), axis=1, keepdims=True)
    h = jnp.concatenate([feats, fusion], axis=1)
    h = jax.nn.relu(h @ pred1_W + pred1_b)
    h = jax.nn.relu(h @ pred2_W + pred2_b)
    out = h @ pred3_W + pred3_b
    return out


# speedup vs baseline: 3.9001x; 3.9001x over previous
import functools

import jax
import jax.numpy as jnp
from jax import lax
from jax.experimental import pallas as pl
from jax.experimental.pallas import tpu as pltpu
from jax.experimental.pallas import tpu_sc as plsc

N = 10000
E = 320000
NC = 2
NS = 16
NT = NC * NS
NPT = 313
NPAD = NT * NPT
ACC_ROWS = 320
DUMMY = 316
CAP = 12800
ECHUNK = 2560
GCHUNK = 512

_MESH = plsc.VectorSubcoreMesh(core_axis_name="c", subcore_axis_name="s")


def _wid():
    return lax.axis_index("s") * NC + lax.axis_index("c")


def _bucket_body(ei_hbm, srcs_hbm, dstls_hbm, cnts_hbm,
                 src_v, dst_v, sbuf, dbuf, cnt_v):
    wid = _wid()
    lo = wid * NPT
    hi = lo + NPT
    zeros16 = jnp.zeros((16,), jnp.int32)
    dummy16 = jnp.full((16,), DUMMY, jnp.int32)

    def init_body(i, c):
        sbuf[pl.ds(i * 16, 16)] = zeros16
        dbuf[pl.ds(i * 16, 16)] = dummy16
        return c
    lax.fori_loop(0, CAP // 16, init_body, 0)

    def scan_body(g, off):
        base = g * ECHUNK
        pltpu.sync_copy(ei_hbm.at[0, pl.ds(base, ECHUNK)], src_v)
        pltpu.sync_copy(ei_hbm.at[1, pl.ds(base, ECHUNK)], dst_v)

        def grp(j, off):
            d = dst_v[pl.ds(j * 16, 16)]
            s = src_v[pl.ds(j * 16, 16)]
            m = (d >= lo) & (d < hi)
            offc = jnp.minimum(off, CAP - 16)
            mi = m.astype(jnp.int32)
            pos = jnp.where(m, offc + jnp.cumsum(mi) - 1, CAP - 1)
            plsc.store_scatter(sbuf, [pos], s)
            plsc.store_scatter(dbuf, [pos], d - lo)
            return off + jnp.sum(mi)
        return lax.fori_loop(0, ECHUNK // 16, grp, off)

    off = lax.fori_loop(0, E // ECHUNK, scan_body, jnp.int32(0))

    offc = jnp.minimum(off, CAP - 16)
    sbuf[pl.ds(offc, 16)] = zeros16
    dbuf[pl.ds(offc, 16)] = dummy16
    cnt_pad = jnp.minimum((off + 15) & (-16), CAP)
    cnt_v[...] = jnp.full((16,), 0, jnp.int32) + cnt_pad
    pltpu.sync_copy(sbuf, srcs_hbm.at[wid])
    pltpu.sync_copy(dbuf, dstls_hbm.at[wid])
    pltpu.sync_copy(cnt_v, cnts_hbm.at[wid])


_bucket = pl.kernel(
    _bucket_body,
    out_type=[jax.ShapeDtypeStruct((NT, CAP), jnp.int32),
              jax.ShapeDtypeStruct((NT, CAP), jnp.int32),
              jax.ShapeDtypeStruct((NT, 16), jnp.int32)],
    mesh=_MESH,
    scratch_types=[pltpu.VMEM((ECHUNK,), jnp.int32),
                   pltpu.VMEM((ECHUNK,), jnp.int32),
                   pltpu.VMEM((CAP,), jnp.int32),
                   pltpu.VMEM((CAP,), jnp.int32),
                   pltpu.VMEM((16,), jnp.int32)],
    compiler_params=pltpu.CompilerParams(needs_layout_passes=False),
    name="edge_bucket",
)


def _segmax_body(C, x_hbm, srcs_hbm, dstls_hbm, cnts_hbm, out_hbm,
                 src_v, dstl_v, rows_v, acc, cnt_v, sem):
    wid = _wid()
    pltpu.sync_copy(cnts_hbm.at[wid], cnt_v)
    cnt = cnt_v[pl.ds(0, 16)][0]

    ninf = jnp.full((16,), -jnp.inf, jnp.float32)

    def init_body(i, c):
        acc[pl.ds(i * 16, 16)] = ninf
        return c
    lax.fori_loop(0, ACC_ROWS * C // 16, init_body, 0)

    nchunks = (cnt + GCHUNK - 1) // GCHUNK

    def chunk_body(g, c):
        base = g * GCHUNK
        pltpu.sync_copy(srcs_hbm.at[wid, pl.ds(base, GCHUNK)], src_v)
        pltpu.sync_copy(dstls_hbm.at[wid, pl.ds(base, GCHUNK)], dstl_v)
        cps = [pltpu.async_copy(x_hbm.at[src_v.at[pl.ds(j * 128, 128)]],
                                rows_v.at[pl.ds(j * 128, 128)], sem)
               for j in range(GCHUNK // 128)]
        for cp in cps:
            cp.wait()
        n_e = jnp.minimum(GCHUNK, cnt - base)

        def ebody(e16, c):
            dv = dstl_v[pl.ds(e16 * 16, 16)]
            for j in range(16):
                e = e16 * 16 + j
                rowbase = dv[j] * C
                for k in range(C // 16):
                    a = acc[pl.ds(rowbase + k * 16, 16)]
                    r = rows_v[e, pl.ds(k * 16, 16)]
                    acc[pl.ds(rowbase + k * 16, 16)] = jnp.maximum(a, r)
            return c
        lax.fori_loop(0, n_e // 16, ebody, 0)
        return c
    lax.fori_loop(0, nchunks, chunk_body, 0)

    pltpu.sync_copy(acc.at[pl.ds(0, NPT * C)],
                    out_hbm.at[pl.ds(wid * NPT * C, NPT * C)])


def _make_segmax(C):
    return pl.kernel(
        functools.partial(_segmax_body, C),
        out_type=jax.ShapeDtypeStruct((NPAD * C,), jnp.float32),
        mesh=_MESH,
        scratch_types=[pltpu.VMEM((GCHUNK,), jnp.int32),
                       pltpu.VMEM((GCHUNK,), jnp.int32),
                       pltpu.VMEM((GCHUNK, C), jnp.float32),
                       pltpu.VMEM((ACC_ROWS * C,), jnp.float32),
                       pltpu.VMEM((16,), jnp.int32),
                       pltpu.SemaphoreType.DMA],
        compiler_params=pltpu.CompilerParams(needs_layout_passes=False,
                                             use_tc_tiling_on_sc=False),
        name=f"segmax{C}",
    )


_segmax128 = _make_segmax(128)
_segmax64 = _make_segmax(64)


def _layer_call(C, residual, x, m, W1, W2, b):
    BN = 1000

    def body(x_ref, m_ref, w1_ref, w2_ref, b_ref, o_ref):
        xb = x_ref[...]
        mb = m_ref[...]
        agg = jnp.where(mb == -jnp.inf, 0.0, mb - xb)
        h = (jnp.dot(xb, w1_ref[...], preferred_element_type=jnp.float32)
             + jnp.dot(agg, w2_ref[...], preferred_element_type=jnp.float32)
             + b_ref[...])
        h = jnp.maximum(h, 0.0)
        if residual:
            h = h + xb
        o_ref[...] = h

    return pl.pallas_call(
        body,
        grid=(N // BN,),
        in_specs=[pl.BlockSpec((BN, C), lambda i: (i, 0)),
                  pl.BlockSpec((BN, C), lambda i: (i, 0)),
                  pl.BlockSpec((C, 64), lambda i: (0, 0)),
                  pl.BlockSpec((C, 64), lambda i: (0, 0)),
                  pl.BlockSpec((1, 64), lambda i: (0, 0))],
        out_specs=pl.BlockSpec((BN, 64), lambda i: (i, 0)),
        out_shape=jax.ShapeDtypeStruct((N, 64), jnp.float32),
    )(x, m, W1, W2, b)


def _tail_call(feats, fusion_W, fusion_b, p1f, p1v, p1b, W2, b2, W3, b3):
    BN = 400

    def body(f_ref, fw_ref, fb_ref, p1f_ref, p1v_ref, p1b_ref,
             w2_ref, b2_ref, w3_ref, b3_ref, o_ref):
        fb = f_ref[...]
        t = jnp.maximum(
            jnp.dot(fb, fw_ref[...], preferred_element_type=jnp.float32)
            + fb_ref[...], 0.0)
        fu = jnp.max(t, axis=1, keepdims=True)
        h1 = jnp.maximum(
            jnp.dot(fb, p1f_ref[...], preferred_element_type=jnp.float32)
            + fu * p1v_ref[...] + p1b_ref[...], 0.0)
        h2 = jnp.maximum(
            jnp.dot(h1, w2_ref[...], preferred_element_type=jnp.float32)
            + b2_ref[...], 0.0)
        o_ref[...] = (jnp.dot(h2, w3_ref[...], preferred_element_type=jnp.float32)
                      + b3_ref[...])

    F = feats.shape[1]
    return pl.pallas_call(
        body,
        grid=(N // BN,),
        in_specs=[pl.BlockSpec((BN, F), lambda i: (i, 0)),
                  pl.BlockSpec((F, 1024), lambda i: (0, 0)),
                  pl.BlockSpec((1, 1024), lambda i: (0, 0)),
                  pl.BlockSpec((F, 512), lambda i: (0, 0)),
                  pl.BlockSpec((1, 512), lambda i: (0, 0)),
                  pl.BlockSpec((1, 512), lambda i: (0, 0)),
                  pl.BlockSpec((512, 256), lambda i: (0, 0)),
                  pl.BlockSpec((1, 256), lambda i: (0, 0)),
                  pl.BlockSpec((256, 13), lambda i: (0, 0)),
                  pl.BlockSpec((1, 13), lambda i: (0, 0))],
        out_specs=pl.BlockSpec((BN, 13), lambda i: (i, 0)),
        out_shape=jax.ShapeDtypeStruct((N, 13), jnp.float32),
    )(feats, fusion_W, fusion_b, p1f, p1v, p1b, W2, b2, W3, b3)


def kernel(x, edge_index, batch, head_W, head_b, blocks_W, blocks_b,
           fusion_W, fusion_b, pred1_W, pred1_b, pred2_W, pred2_b,
           pred3_W, pred3_b):
    srcs, dstls, cnts = _bucket(edge_index)

    m0 = _segmax128(x, srcs, dstls, cnts).reshape(NPAD, 128)[:N]
    h = _layer_call(128, False, x, m0, head_W[:128], head_W[128:],
                    head_b.reshape(1, 64))
    feats = [h]
    for i in range(6):
        m = _segmax64(h, srcs, dstls, cnts).reshape(NPAD, 64)[:N]
        h = _layer_call(64, True, h, m, blocks_W[i, :64], blocks_W[i, 64:],
                        blocks_b[i].reshape(1, 64))
        feats.append(h)
    feats = jnp.concatenate(feats, axis=1)

    return _tail_call(feats, fusion_W, fusion_b.reshape(1, 1024),
                      pred1_W[:448], pred1_W[448:449], pred1_b.reshape(1, 512),
                      pred2_W, pred2_b.reshape(1, 256),
                      pred3_W, pred3_b.reshape(1, 13))
